# Initial kernel scaffold; baseline (speedup 1.0000x reference)
import jax
import jax.numpy as jnp
from jax.experimental import pallas as pl


def kernel(x, pos, neighbor_idx, W_in, b_in, Wp1, bp1, Wp2, bp2, W_out, b_out):
    raise NotImplementedError("write your pallas kernel here")


# speedup vs baseline: 22.6081x; 22.6081x over previous
import functools

import jax
import jax.numpy as jnp
from jax import lax
from jax.experimental import pallas as pl
from jax.experimental.pallas import tpu as pltpu
from jax.experimental.pallas import tpu_sc as plsc

_SLOPE = 0.1
_PAD_TO = 16
_GATHER_WINDOW = 128


def _leaky(v):
    return jnp.where(v >= 0, v, _SLOPE * v)


def _table_body(x_ref, pos_ref, w_ref, b_ref, o_ref):
    h = jnp.dot(x_ref[...], w_ref[...], preferred_element_type=jnp.float32)
    h = _leaky(h + b_ref[...])
    p = pos_ref[...]
    pad = jnp.zeros((p.shape[0], o_ref.shape[1] - h.shape[1] - p.shape[1]),
                    jnp.float32)
    o_ref[...] = jnp.concatenate([h, p, pad], axis=1)


def _build_table(x2, pos2, W_in, b_in, block_rows):
    BN, C_in = x2.shape
    H = W_in.shape[1]
    D = H + _PAD_TO
    grid = (BN // block_rows,)
    return pl.pallas_call(
        _table_body,
        grid=grid,
        in_specs=[
            pl.BlockSpec((block_rows, C_in), lambda i: (i, 0)),
            pl.BlockSpec((block_rows, 3), lambda i: (i, 0)),
            pl.BlockSpec((C_in, H), lambda i: (0, 0)),
            pl.BlockSpec((1, H), lambda i: (0, 0)),
        ],
        out_specs=pl.BlockSpec((block_rows, 128), lambda i: (i, 0)),
        out_shape=jax.ShapeDtypeStruct((BN, 128), jnp.float32),
    )(x2, pos2, W_in, b_in.reshape(1, H))


def _sc_gather(table, gidx_flat):
    E = gidx_flat.shape[1]
    D = table.shape[1]
    W = _GATHER_WINDOW
    mesh = plsc.VectorSubcoreMesh(core_axis_name="c", subcore_axis_name="s")

    @functools.partial(
        pl.kernel,
        out_type=jax.ShapeDtypeStruct((E, D), jnp.float32),
        mesh=mesh,
    )
    def gather_kernel(tbl_hbm, idx_hbm, out_hbm):
        def body(i_vmem, o_vmem):
            pltpu.sync_copy(tbl_hbm.at[i_vmem.at[0]], o_vmem)

        pltpu.emit_pipeline(
            body,
            grid=(E // W,),
            in_specs=[pl.BlockSpec((1, W), lambda i: (0, i))],
            out_specs=[pl.BlockSpec((W, D), lambda i: (i, 0))],
            core_axis_name=("c", "s"),
            dimension_semantics=(pltpu.PARALLEL,),
        )(idx_hbm, out_hbm)

    return gather_kernel(table, gidx_flat)


def _combine_body(g_ref, pos_ref, x_ref, w1_ref, b1_ref, w2_ref, b2_ref,
                  wo_ref, bo_ref, o_ref):
    K = g_ref.shape[0]
    H = w2_ref.shape[0]
    posb = pos_ref[...]
    P = posb.shape[0]
    w1 = w1_ref[...]
    b1 = b1_ref[...]
    w2 = w2_ref[...]
    b2 = b2_ref[...]
    acc = jnp.zeros((P, H), jnp.float32)
    for k in range(K):
        gk = g_ref[k]
        rel = posb - gk[:, H:H + 3]
        t = (rel[:, 0:1] * w1[0:1, :]
             + rel[:, 1:2] * w1[1:2, :]
             + rel[:, 2:3] * w1[2:3, :]
             + b1)
        t = _leaky(t)
        wk = jnp.dot(t, w2, preferred_element_type=jnp.float32) + b2
        acc = acc + wk * gk[:, :H]
    out = jnp.dot(acc, wo_ref[...], preferred_element_type=jnp.float32)
    o_ref[...] = _leaky(out + bo_ref[...] + x_ref[...])


def _combine(gath3, pos2, x2, Wp1, bp1, Wp2, bp2, W_out, b_out, block_rows):
    K, BN, D = gath3.shape
    H = Wp2.shape[0]
    C_out = W_out.shape[1]
    C_in = x2.shape[1]
    grid = (BN // block_rows,)
    return pl.pallas_call(
        _combine_body,
        grid=grid,
        in_specs=[
            pl.BlockSpec((K, block_rows, D), lambda i: (0, i, 0)),
            pl.BlockSpec((block_rows, 3), lambda i: (i, 0)),
            pl.BlockSpec((block_rows, C_in), lambda i: (i, 0)),
            pl.BlockSpec((3, H), lambda i: (0, 0)),
            pl.BlockSpec((1, H), lambda i: (0, 0)),
            pl.BlockSpec((H, H), lambda i: (0, 0)),
            pl.BlockSpec((1, H), lambda i: (0, 0)),
            pl.BlockSpec((H, C_out), lambda i: (0, 0)),
            pl.BlockSpec((1, C_out), lambda i: (0, 0)),
        ],
        out_specs=pl.BlockSpec((block_rows, C_out), lambda i: (i, 0)),
        out_shape=jax.ShapeDtypeStruct((BN, C_out), jnp.float32),
    )(gath3, pos2, x2, Wp1, bp1.reshape(1, H), Wp2, bp2.reshape(1, H),
      W_out, b_out.reshape(1, C_out))


def kernel(x, pos, neighbor_idx, W_in, b_in, Wp1, bp1, Wp2, bp2, W_out, b_out):
    B, N, C_in = x.shape
    K = neighbor_idx.shape[2]
    H = W_in.shape[1]
    D = H + _PAD_TO
    BN = B * N
    E = BN * K

    x2 = x.reshape(BN, C_in)
    pos2 = pos.reshape(BN, 3)

    table = _build_table(x2, pos2, W_in, b_in, block_rows=400)

    offs = (jnp.arange(B, dtype=jnp.int32) * N)[:, None, None]
    gidx = jnp.transpose(neighbor_idx + offs, (2, 0, 1)).reshape(1, E)

    gath = _sc_gather(table, gidx).reshape(K, BN, 128)

    out2 = _combine(gath, pos2, x2, Wp1, bp1, Wp2, bp2, W_out, b_out,
                    block_rows=400)
    return out2.reshape(B, N, W_out.shape[1])


# speedup vs baseline: 39.4496x; 1.7449x over previous
import functools

import jax
import jax.numpy as jnp
from jax import lax
from jax.experimental import pallas as pl
from jax.experimental.pallas import tpu as pltpu
from jax.experimental.pallas import tpu_sc as plsc

_SLOPE = 0.1
_PAD_TO = 16
_GATHER_WINDOW = 128


def _leaky(v):
    return jnp.where(v >= 0, v, _SLOPE * v)


def _table_body(x_ref, pos_ref, w_ref, b_ref, o_ref):
    h = jnp.dot(x_ref[...], w_ref[...], preferred_element_type=jnp.float32)
    h = _leaky(h + b_ref[...])
    p = pos_ref[...]
    pad = jnp.zeros((p.shape[0], o_ref.shape[1] - h.shape[1] - p.shape[1]),
                    jnp.float32)
    o_ref[...] = jnp.concatenate([h, p, pad], axis=1)


def _build_table(x2, pos2, W_in, b_in, block_rows):
    BN, C_in = x2.shape
    H = W_in.shape[1]
    D = H + _PAD_TO
    grid = (BN // block_rows,)
    return pl.pallas_call(
        _table_body,
        grid=grid,
        in_specs=[
            pl.BlockSpec((block_rows, C_in), lambda i: (i, 0)),
            pl.BlockSpec((block_rows, 3), lambda i: (i, 0)),
            pl.BlockSpec((C_in, H), lambda i: (0, 0)),
            pl.BlockSpec((1, H), lambda i: (0, 0)),
        ],
        out_specs=pl.BlockSpec((block_rows, 128), lambda i: (i, 0)),
        out_shape=jax.ShapeDtypeStruct((BN, 128), jnp.float32),
    )(x2, pos2, W_in, b_in.reshape(1, H))


def _sc_gather(table, gidx_flat):
    E = gidx_flat.shape[1]
    D = table.shape[1]
    W = _GATHER_WINDOW
    mesh = plsc.VectorSubcoreMesh(core_axis_name="c", subcore_axis_name="s")

    @functools.partial(
        pl.kernel,
        out_type=jax.ShapeDtypeStruct((E, D), jnp.float32),
        mesh=mesh,
    )
    def gather_kernel(tbl_hbm, idx_hbm, out_hbm):
        def body(i_vmem, o_vmem):
            pltpu.sync_copy(tbl_hbm.at[i_vmem.at[0]], o_vmem)

        pltpu.emit_pipeline(
            body,
            grid=(E // W,),
            in_specs=[pl.BlockSpec((1, W), lambda i: (0, i))],
            out_specs=[pl.BlockSpec((W, D), lambda i: (i, 0))],
            core_axis_name=("c", "s"),
            dimension_semantics=(pltpu.PARALLEL,),
        )(idx_hbm, out_hbm)

    return gather_kernel(table, gidx_flat)


def _combine_body(g_ref, pos_ref, x_ref, w1_ref, b1_ref, w2_ref, b2_ref,
                  wo_ref, bo_ref, o_ref):
    K = g_ref.shape[0]
    H = w2_ref.shape[0]
    posb = pos_ref[...]
    P = posb.shape[0]
    w1 = w1_ref[...]
    b1 = b1_ref[...]
    w2 = w2_ref[...]
    b2 = b2_ref[...]
    acc = jnp.zeros((P, H), jnp.float32)
    for k in range(K):
        gk = g_ref[k]
        rel = posb - gk[:, H:H + 3]
        t = jnp.dot(rel, w1, preferred_element_type=jnp.float32) + b1
        t = _leaky(t)
        wk = jnp.dot(t, w2, preferred_element_type=jnp.float32) + b2
        acc = acc + wk * gk[:, :H]
    out = jnp.dot(acc, wo_ref[...], preferred_element_type=jnp.float32)
    o_ref[...] = _leaky(out + bo_ref[...] + x_ref[...])


def _combine(gath3, pos2, x2, Wp1, bp1, Wp2, bp2, W_out, b_out, block_rows):
    K, BN, D = gath3.shape
    H = Wp2.shape[0]
    C_out = W_out.shape[1]
    C_in = x2.shape[1]
    grid = (BN // block_rows,)
    return pl.pallas_call(
        _combine_body,
        grid=grid,
        in_specs=[
            pl.BlockSpec((K, block_rows, D), lambda i: (0, i, 0)),
            pl.BlockSpec((block_rows, 3), lambda i: (i, 0)),
            pl.BlockSpec((block_rows, C_in), lambda i: (i, 0)),
            pl.BlockSpec((3, H), lambda i: (0, 0)),
            pl.BlockSpec((1, H), lambda i: (0, 0)),
            pl.BlockSpec((H, H), lambda i: (0, 0)),
            pl.BlockSpec((1, H), lambda i: (0, 0)),
            pl.BlockSpec((H, C_out), lambda i: (0, 0)),
            pl.BlockSpec((1, C_out), lambda i: (0, 0)),
        ],
        out_specs=pl.BlockSpec((block_rows, C_out), lambda i: (i, 0)),
        out_shape=jax.ShapeDtypeStruct((BN, C_out), jnp.float32),
    )(gath3, pos2, x2, Wp1, bp1.reshape(1, H), Wp2, bp2.reshape(1, H),
      W_out, b_out.reshape(1, C_out))


def kernel(x, pos, neighbor_idx, W_in, b_in, Wp1, bp1, Wp2, bp2, W_out, b_out):
    B, N, C_in = x.shape
    K = neighbor_idx.shape[2]
    H = W_in.shape[1]
    D = H + _PAD_TO
    BN = B * N
    E = BN * K

    x2 = x.reshape(BN, C_in)
    pos2 = pos.reshape(BN, 3)

    table = _build_table(x2, pos2, W_in, b_in, block_rows=2000)

    offs = (jnp.arange(B, dtype=jnp.int32) * N)[:, None, None]
    gidx = jnp.transpose(neighbor_idx + offs, (2, 0, 1)).reshape(1, E)

    gath = _sc_gather(table, gidx).reshape(K, BN, 128)

    out2 = _combine(gath, pos2, x2, Wp1, bp1, Wp2, bp2, W_out, b_out,
                    block_rows=400)
    return out2.reshape(B, N, W_out.shape[1])
